# baseline (speedup 1.0000x reference)
import functools

import jax
import jax.numpy as jnp
from jax import lax
from jax.experimental import pallas as pl
from jax.experimental.pallas import tpu as pltpu
from jax.experimental.pallas import tpu_sc as plsc

BATCH = 16384
EMBED = 32
USER_N = 1000000
ITEM_N = 1000000
C1_NUM = 10000
C2_NUM = 100
NC = 2
NS = 16
NW = NC * NS
BPW = BATCH // NW
IC = 128
NCHUNK = BPW // IC
LANES = 16
NKC = BPW // LANES
SB = 128
NSB = BPW // SB
SBI = SB * EMBED
KC_PER_SB = SB // LANES


def _sc_body(uid_hbm, iid_hbm, ut_hbm, it_hbm, ct1_hbm, ct2_hbm, a1_hbm, a2_hbm,
             out_hbm,
             uid_v, iid_v, c1_v, u_v, b_v, l1_v, ct2_v, a2_v, out_v,
             ixu_a, ixb_a, ixl_a, ixu_b, ixb_b, ixl_b,
             sem_a, sem_b, sem_c1, sem_t):
    wid = lax.axis_index("s") * NC + lax.axis_index("c")
    base = wid * BPW

    pltpu.sync_copy(uid_hbm.at[pl.ds(base, BPW)], uid_v)
    pltpu.sync_copy(iid_hbm.at[pl.ds(base, BPW)], iid_v)

    t1 = pltpu.async_copy(a2_hbm, a2_v, sem_t)
    t2 = pltpu.async_copy(ct2_hbm, ct2_v, sem_t)

    hc = []
    for j in range(NCHUNK):
        s = pl.ds(j * IC, IC)
        hc.append(pltpu.async_copy(a1_hbm.at[iid_v.at[s]], c1_v.at[s], sem_c1))
    for h in hc:
        h.wait()

    def build(sb, ixu, ixb, ixl):
        def bld(kc, carry):
            k = sb * SB + kc * LANES
            u16 = uid_v[pl.ds(k, LANES)]
            i16 = iid_v[pl.ds(k, LANES)]
            c16 = c1_v[pl.ds(k, LANES)]
            for d in range(EMBED):
                off = kc * (EMBED * LANES) + d * LANES
                ixu[pl.ds(off, LANES)] = u16 + d * USER_N
                ixb[pl.ds(off, LANES)] = i16 + d * ITEM_N
                ixl[pl.ds(off, LANES)] = c16 + d * C1_NUM
            return carry
        lax.fori_loop(0, KC_PER_SB, bld, 0, unroll=False)

    def fire(sb, ixu, ixb, ixl, sem):
        for j in range(SBI // IC):
            s = pl.ds(j * IC, IC)
            dsts = pl.ds(sb * SBI + j * IC, IC)
            pltpu.async_copy(ut_hbm.at[ixu.at[s]], u_v.at[dsts], sem)
            pltpu.async_copy(it_hbm.at[ixb.at[s]], b_v.at[dsts], sem)
            pltpu.async_copy(ct1_hbm.at[ixl.at[s]], l1_v.at[dsts], sem)

    def drain(sem):
        for _ in range(SBI // IC):
            pltpu.make_async_copy(ut_hbm.at[pl.ds(0, IC)],
                                  u_v.at[pl.ds(0, IC)], sem).wait()
            pltpu.make_async_copy(it_hbm.at[pl.ds(0, IC)],
                                  b_v.at[pl.ds(0, IC)], sem).wait()
            pltpu.make_async_copy(ct1_hbm.at[pl.ds(0, IC)],
                                  l1_v.at[pl.ds(0, IC)], sem).wait()

    build(0, ixu_a, ixb_a, ixl_a)
    fire(0, ixu_a, ixb_a, ixl_a, sem_a)
    build(1, ixu_b, ixb_b, ixl_b)
    fire(1, ixu_b, ixb_b, ixl_b, sem_b)
    drain(sem_a)
    build(2, ixu_a, ixb_a, ixl_a)
    fire(2, ixu_a, ixb_a, ixl_a, sem_a)
    drain(sem_b)
    build(3, ixu_b, ixb_b, ixl_b)
    fire(3, ixu_b, ixb_b, ixl_b, sem_b)
    drain(sem_a)
    drain(sem_b)
    t1.wait()
    t2.wait()

    iota = lax.iota(jnp.int32, LANES)

    def body(g, carry):
        c1 = c1_v[pl.ds(g * LANES, LANES)]
        c2 = plsc.load_gather(a2_v, [c1])
        acc = jnp.zeros((LANES,), jnp.float32)
        for d in range(EMBED):
            off = g * (EMBED * LANES) + d * LANES
            v = (b_v[pl.ds(off, LANES)]
                 + l1_v[pl.ds(off, LANES)]
                 + plsc.load_gather(ct2_v, [c2 + d * C2_NUM]))
            acc = acc + u_v[pl.ds(off, LANES)] * v
        plsc.store_scatter(out_v, [g * LANES + iota], acc)
        return carry

    lax.fori_loop(0, NKC, body, 0)
    pltpu.sync_copy(out_v, out_hbm.at[pl.ds(base, BPW)])


@functools.partial(jax.jit)
def _run(uid, iid, ut_f, it_f, ct1_f, ct2_f, a1, a2):
    mesh = plsc.VectorSubcoreMesh(core_axis_name="c", subcore_axis_name="s")
    k = pl.kernel(
        _sc_body,
        mesh=mesh,
        compiler_params=pltpu.CompilerParams(
            needs_layout_passes=False, use_tc_tiling_on_sc=False),
        out_type=jax.ShapeDtypeStruct((BATCH,), jnp.float32),
        scratch_types=[
            pltpu.VMEM((BPW,), jnp.int32),
            pltpu.VMEM((BPW,), jnp.int32),
            pltpu.VMEM((BPW,), jnp.int32),
            pltpu.VMEM((BPW * EMBED,), jnp.float32),
            pltpu.VMEM((BPW * EMBED,), jnp.float32),
            pltpu.VMEM((BPW * EMBED,), jnp.float32),
            pltpu.VMEM((C2_NUM * EMBED,), jnp.float32),
            pltpu.VMEM((C1_NUM,), jnp.int32),
            pltpu.VMEM((BPW,), jnp.float32),
            pltpu.VMEM((SBI,), jnp.int32),
            pltpu.VMEM((SBI,), jnp.int32),
            pltpu.VMEM((SBI,), jnp.int32),
            pltpu.VMEM((SBI,), jnp.int32),
            pltpu.VMEM((SBI,), jnp.int32),
            pltpu.VMEM((SBI,), jnp.int32),
            pltpu.SemaphoreType.DMA,
            pltpu.SemaphoreType.DMA,
            pltpu.SemaphoreType.DMA,
            pltpu.SemaphoreType.DMA,
        ],
    )
    return k(uid, iid, ut_f, it_f, ct1_f, ct2_f, a1, a2)


def kernel(X, user_table, item_table, cluster_table1, cluster_table2, assign1, assign2):
    uid = X[:, 0].astype(jnp.int32)
    iid = X[:, 1].astype(jnp.int32)
    out = _run(uid, iid,
               user_table.T.reshape(-1), item_table.T.reshape(-1),
               cluster_table1.T.reshape(-1), cluster_table2.T.reshape(-1),
               assign1.astype(jnp.int32), assign2.astype(jnp.int32))
    return out.reshape(BATCH, 1)


# speedup vs baseline: 5.5485x; 5.5485x over previous
import functools

import jax
import jax.numpy as jnp
from jax import lax
from jax.experimental import pallas as pl
from jax.experimental.pallas import tpu as pltpu
from jax.experimental.pallas import tpu_sc as plsc

BATCH = 16384
EMBED = 32
C1_NUM = 10000
C2_NUM = 100
NC = 2
NS = 16
NW = NC * NS
BPW = BATCH // NW
IC = 128
NCHUNK = BPW // IC
LANES = 16


def _sc_body(uid_hbm, iid_hbm, ut_hbm, it_hbm, ct1_hbm, ct2_hbm, a1_hbm, a2_hbm,
             out_hbm,
             uid_v, iid_v, c1_v, u_v, b_v, l1_v, ct2_v, a2_v, out_v,
             sem_u, sem_b, sem_c1, sem_l1, sem_t):
    wid = lax.axis_index("s") * NC + lax.axis_index("c")
    base = wid * BPW

    pltpu.sync_copy(uid_hbm.at[pl.ds(base, BPW)], uid_v)
    pltpu.sync_copy(iid_hbm.at[pl.ds(base, BPW)], iid_v)

    t1 = pltpu.async_copy(a2_hbm, a2_v, sem_t)
    t2 = pltpu.async_copy(ct2_hbm, ct2_v, sem_t)

    hu, hb, hc = [], [], []
    for j in range(NCHUNK):
        s = pl.ds(j * IC, IC)
        hu.append(pltpu.async_copy(ut_hbm.at[uid_v.at[s]], u_v.at[s], sem_u))
        hb.append(pltpu.async_copy(it_hbm.at[iid_v.at[s]], b_v.at[s], sem_b))
        hc.append(pltpu.async_copy(a1_hbm.at[iid_v.at[s]], c1_v.at[s], sem_c1))
    for h in hc:
        h.wait()
    hl = []
    for j in range(NCHUNK):
        s = pl.ds(j * IC, IC)
        hl.append(pltpu.async_copy(ct1_hbm.at[c1_v.at[s]], l1_v.at[s], sem_l1))
    t1.wait()
    t2.wait()
    for h in hu:
        h.wait()
    for h in hb:
        h.wait()
    for h in hl:
        h.wait()

    iota = lax.iota(jnp.int32, LANES)

    def body(g, carry):
        rows = g * LANES + iota
        c1 = plsc.load_gather(c1_v, [rows])
        c2 = plsc.load_gather(a2_v, [c1])
        acc = jnp.zeros((LANES,), jnp.float32)
        for d in range(EMBED):
            cold = jnp.full((LANES,), d, jnp.int32)
            u = plsc.load_gather(u_v, [rows, cold])
            v = (plsc.load_gather(b_v, [rows, cold])
                 + plsc.load_gather(l1_v, [rows, cold])
                 + plsc.load_gather(ct2_v, [c2, cold]))
            acc = acc + u * v
        plsc.store_scatter(out_v, [rows], acc)
        return carry

    lax.fori_loop(0, BPW // LANES, body, 0)
    pltpu.sync_copy(out_v, out_hbm.at[pl.ds(base, BPW)])


@functools.partial(jax.jit)
def _run(uid, iid, ut, it, ct1, ct2, a1, a2):
    mesh = plsc.VectorSubcoreMesh(core_axis_name="c", subcore_axis_name="s")
    k = pl.kernel(
        _sc_body,
        mesh=mesh,
        compiler_params=pltpu.CompilerParams(
            needs_layout_passes=False, use_tc_tiling_on_sc=False),
        out_type=jax.ShapeDtypeStruct((BATCH,), jnp.float32),
        scratch_types=[
            pltpu.VMEM((BPW,), jnp.int32),
            pltpu.VMEM((BPW,), jnp.int32),
            pltpu.VMEM((BPW,), jnp.int32),
            pltpu.VMEM((BPW, EMBED), jnp.float32),
            pltpu.VMEM((BPW, EMBED), jnp.float32),
            pltpu.VMEM((BPW, EMBED), jnp.float32),
            pltpu.VMEM((C2_NUM, EMBED), jnp.float32),
            pltpu.VMEM((C1_NUM,), jnp.int32),
            pltpu.VMEM((BPW,), jnp.float32),
            pltpu.SemaphoreType.DMA,
            pltpu.SemaphoreType.DMA,
            pltpu.SemaphoreType.DMA,
            pltpu.SemaphoreType.DMA,
            pltpu.SemaphoreType.DMA,
        ],
    )
    return k(uid, iid, ut, it, ct1, ct2, a1, a2)


def kernel(X, user_table, item_table, cluster_table1, cluster_table2, assign1, assign2):
    uid = X[:, 0].astype(jnp.int32)
    iid = X[:, 1].astype(jnp.int32)
    out = _run(uid, iid, user_table, item_table, cluster_table1, cluster_table2,
               assign1.astype(jnp.int32), assign2.astype(jnp.int32))
    return out.reshape(BATCH, 1)
